# baseline (speedup 1.0000x reference)
import jax
import jax.numpy as jnp
from jax.experimental import pallas as pl
from jax.experimental.pallas import tpu as pltpu

L = 512
WORD = 512
D = 256
NT = 12
TP = 128
W = 8
START = 10
NSP = 64
NEG = -1e30

BL = 264


def _dot(x, wt):
    return jax.lax.dot_general(
        x.astype(jnp.bfloat16), wt.astype(jnp.bfloat16), (((1,), (0,)), ((), ())),
        preferred_element_type=jnp.float32)


def _band_kernel(feats_ref, dense_w_ref, dense_b_ref, wl_w_ref, wl_b_ref,
                 wr_w_ref, wr_b_ref, gl_w_ref, gl_b_ref, gr_w_ref, gr_b_ref,
                 tag_w_ref, tag_b_ref, emitb_ref):
    pid = pl.program_id(0)
    x = feats_ref[pl.ds(pid * 256, BL), :]
    cur = _dot(x, dense_w_ref[...]) + dense_b_ref[...]
    for s in range(W):
        if s > 0:
            hl = cur
            hr = pltpu.roll(cur, BL - 1, axis=0)
            pre = (_dot(hl, wl_w_ref[...]) + wl_b_ref[...]
                   + _dot(hr, wr_w_ref[...]) + wr_b_ref[...])
            hhat = 4.0 * jax.nn.sigmoid(pre) - 2.0
            g = (_dot(hl, gl_w_ref[...]) + gl_b_ref[...]
                 + _dot(hr, gr_w_ref[...]) + gr_b_ref[...])
            g0 = g[:, 0:D]
            g1 = g[:, D:2 * D]
            g2 = g[:, 2 * D:3 * D]
            m = jnp.maximum(jnp.maximum(g0, g1), g2)
            e0 = jnp.exp(g0 - m)
            e1 = jnp.exp(g1 - m)
            e2 = jnp.exp(g2 - m)
            inv = 1.0 / (e0 + e1 + e2)
            cur = (e0 * hhat + e1 * hl + e2 * hr) * inv
        emit = _dot(cur, tag_w_ref[...]) + tag_b_ref[...]
        emitb_ref[s, :, :] = emit[0:256, :]


def _scan_kernel(emitb_ref, trans_ref, tagb_ref, tags_ref, out_ref, bnd_ref,
                 mb2_ref, bufres_ref):
    lane = jax.lax.broadcasted_iota(jnp.int32, (1, TP), 1)
    negmask = jnp.where(lane < NT, 0.0, NEG)
    rows_pc = jax.lax.broadcasted_iota(jnp.int32, (TP, TP), 0)
    cols_pc = jax.lax.broadcasted_iota(jnp.int32, (TP, TP), 1)
    expT = jnp.where((rows_pc < NT) & (cols_pc < NT),
                     jnp.exp(trans_ref[...]), 0.0)
    expTb = expT.astype(jnp.bfloat16)

    def _dotb(v):
        return jax.lax.dot_general(
            v.astype(jnp.bfloat16), expTb, (((1,), (0,)), ((), ())),
            preferred_element_type=jnp.float32)

    rolls = [pltpu.roll(emitb_ref[s, :, :], s, axis=0) for s in range(W)]
    md = rolls[0]
    for s in range(1, W):
        md = jnp.maximum(md, rolls[s])
    md = md + negmask
    for sh in (64, 32, 16, 8, 4, 2, 1):
        md = jnp.maximum(md, pltpu.roll(md, sh, axis=1))
    mb2 = md + pltpu.roll(md, 1, axis=0) + 4.0
    mb2_ref[...] = mb2
    for s in range(W):
        bnd_ref[:, W - 1 - s:W - s, :] = (
            rolls[s] - mb2 + negmask).reshape(L, 1, TP)

    alpha0 = jnp.where(lane == START, 0.0, -1000.0)
    row8 = jax.lax.broadcasted_iota(jnp.int32, (W, TP), 0)
    buf0 = jnp.where(row8 == W - 1,
                     jnp.broadcast_to(alpha0, (W, TP)),
                     NEG)
    r0 = jnp.zeros((1, 1), jnp.float32)

    def step(j, carry):
        buf, na1, r2, mxhi, mxlo = carry
        r1 = jnp.maximum(jnp.max(na1, keepdims=True), NEG)
        ebm = bnd_ref[pl.ds(j, 1), :, :][0]
        mb2row = mb2_ref[pl.ds(j, 1), :]
        ex = (buf - r2) + ebm
        e = jnp.exp(jnp.minimum(ex, 30.0))
        ev = jnp.sum(e, axis=0, keepdims=True)
        na = (r2 + jnp.log(_dotb(ev))) + mb2row
        mxv = jnp.max(ex, keepdims=True)
        return (jnp.concatenate([buf[1:], na], axis=0), na, r1,
                jnp.maximum(mxhi, mxv), jnp.minimum(mxlo, mxv))

    big = jnp.full((1, 1), 1e30, jnp.float32)
    alpha0m = jnp.where(lane < NT, alpha0, NEG)
    buf_spec, _, _, mxhi, mxlo = jax.lax.fori_loop(
        0, L, step, (buf0, alpha0m, r0, -big, big))
    bufres_ref[...] = buf_spec

    bad = jnp.logical_not(
        jnp.logical_and(mxhi[0, 0] <= 30.0, mxlo[0, 0] >= -50.0))

    @pl.when(bad)
    def _():
        def stepx(j, bufx):
            ebm = bnd_ref[pl.ds(j, 1), :, :][0]
            mb2row = mb2_ref[pl.ds(j, 1), :]
            tmp = bufx + ebm
            m2 = jnp.maximum(jnp.max(tmp, keepdims=True), NEG)
            e2 = jnp.exp(tmp - m2)
            ev2 = jnp.sum(e2, axis=0, keepdims=True)
            na = (m2 + jnp.log(_dotb(ev2))) + mb2row
            return jnp.concatenate([bufx[1:], na], axis=0)

        bufres_ref[...] = jax.lax.fori_loop(0, L, stepx, buf0)

    buf = bufres_ref[...]

    final = buf[W - 1:W, :] + negmask
    mf = jnp.max(final, keepdims=True)
    logz = mf + jnp.log(jnp.sum(jnp.exp(final - mf), keepdims=True))

    def gold_step(k, acc):
        i = tags_ref[0, k]
        j = tags_ref[1, k]
        p = tags_ref[2, k]
        c = tags_ref[3, k]
        s = j - i
        onb = jnp.logical_and(s >= 0, s < W)
        ss = jnp.where(onb, s, 0)
        row = emitb_ref[pl.ds(ss, 1), pl.ds(i, 1), :][0]
        trow = trans_ref[pl.ds(p, 1), :]
        pm = lane == p
        emit_term = jnp.where(jnp.logical_and(pm, onb), row, 0.0)
        tb_term = jnp.where(jnp.logical_and(pm, jnp.logical_not(onb)),
                            tagb_ref[...], 0.0)
        tr_term = jnp.where(lane == c, trow, 0.0)
        return acc + emit_term + tb_term + tr_term

    acc = jax.lax.fori_loop(0, NSP, gold_step, jnp.zeros((1, TP), jnp.float32))
    gold = jnp.sum(acc, keepdims=True)
    out_ref[...] = jnp.broadcast_to(logz - gold, (8, TP))


def kernel(feats, tags, dense_w, dense_b, wl_w, wl_b, wr_w, wr_b,
           gl_w, gl_b, gr_w, gr_b, tag_w, tag_b, transitions):
    f = jnp.pad(feats[0], ((0, 2 * BL - L), (0, 0)))
    tag_w_p = jnp.zeros((TP, D), jnp.float32).at[:NT].set(tag_w)
    tag_b_p = jnp.zeros((1, TP), jnp.float32).at[0, :NT].set(tag_b)
    trans_p = jnp.zeros((TP, TP), jnp.float32).at[:NT, :NT].set(transitions)
    tags2 = tags[0].astype(jnp.int32).T

    def full(a):
        return pl.BlockSpec(a.shape, lambda i: (0,) * a.ndim)

    b2 = lambda b: b[None, :]
    args1 = (f, dense_w.T, b2(dense_b), wl_w.T, b2(wl_b), wr_w.T, b2(wr_b),
             gl_w.T, b2(gl_b), gr_w.T, b2(gr_b), tag_w_p.T, tag_b_p)
    emitb = pl.pallas_call(
        _band_kernel,
        grid=(2,),
        in_specs=[full(a) for a in args1],
        out_specs=pl.BlockSpec((W, 256, TP), lambda i: (0, i, 0)),
        out_shape=jax.ShapeDtypeStruct((W, L, TP), jnp.float32),
        compiler_params=pltpu.CompilerParams(
            dimension_semantics=("parallel",)),
    )(*args1)

    out = pl.pallas_call(
        _scan_kernel,
        in_specs=[
            pl.BlockSpec(memory_space=pltpu.VMEM),
            pl.BlockSpec(memory_space=pltpu.VMEM),
            pl.BlockSpec(memory_space=pltpu.VMEM),
            pl.BlockSpec(memory_space=pltpu.SMEM),
        ],
        out_specs=pl.BlockSpec(memory_space=pltpu.VMEM),
        out_shape=jax.ShapeDtypeStruct((8, TP), jnp.float32),
        scratch_shapes=[pltpu.VMEM((L, W, TP), jnp.float32),
                        pltpu.VMEM((L, TP), jnp.float32),
                        pltpu.VMEM((W, TP), jnp.float32)],
    )(emitb, trans_p, tag_b_p, tags2)
    return out[0, 0]


# speedup vs baseline: 1.0110x; 1.0110x over previous
import jax
import jax.numpy as jnp
from jax.experimental import pallas as pl
from jax.experimental.pallas import tpu as pltpu

L = 512
WORD = 512
D = 256
NT = 12
TP = 128
W = 8
START = 10
NSP = 64
NEG = -1e30

BL = 264


def _dot(x, wt):
    return jax.lax.dot_general(
        x, wt, (((1,), (0,)), ((), ())),
        preferred_element_type=jnp.float32)


def _band_kernel(feats_ref, dense_w_ref, dense_b_ref, wl_w_ref, wl_b_ref,
                 wr_w_ref, wr_b_ref, gl_w_ref, gl_b_ref, gr_w_ref, gr_b_ref,
                 tag_w_ref, tag_b_ref, emitb_ref):
    pid = pl.program_id(0)
    x = feats_ref[pl.ds(pid * 256, BL), :]
    cur = _dot(x, dense_w_ref[...]) + dense_b_ref[...]
    for s in range(W):
        if s > 0:
            hl = cur
            hr = pltpu.roll(cur, BL - 1, axis=0)
            pre = (_dot(hl, wl_w_ref[...]) + wl_b_ref[...]
                   + _dot(hr, wr_w_ref[...]) + wr_b_ref[...])
            hhat = 4.0 * jax.nn.sigmoid(pre) - 2.0
            g = (_dot(hl, gl_w_ref[...]) + gl_b_ref[...]
                 + _dot(hr, gr_w_ref[...]) + gr_b_ref[...])
            g0 = g[:, 0:D]
            g1 = g[:, D:2 * D]
            g2 = g[:, 2 * D:3 * D]
            m = jnp.maximum(jnp.maximum(g0, g1), g2)
            e0 = jnp.exp(g0 - m)
            e1 = jnp.exp(g1 - m)
            e2 = jnp.exp(g2 - m)
            inv = 1.0 / (e0 + e1 + e2)
            cur = (e0 * hhat + e1 * hl + e2 * hr) * inv
        emit = _dot(cur, tag_w_ref[...]) + tag_b_ref[...]
        emitb_ref[s, :, :] = emit[0:256, :]


def _scan_kernel(emitb_ref, trans_ref, tagb_ref, tags_ref, out_ref, bnd_ref,
                 mb2_ref, bufres_ref):
    lane = jax.lax.broadcasted_iota(jnp.int32, (1, TP), 1)
    negmask = jnp.where(lane < NT, 0.0, NEG)
    rows_pc = jax.lax.broadcasted_iota(jnp.int32, (TP, TP), 0)
    cols_pc = jax.lax.broadcasted_iota(jnp.int32, (TP, TP), 1)
    expT = jnp.where((rows_pc < NT) & (cols_pc < NT),
                     jnp.exp(trans_ref[...]), 0.0)
    expTb = expT.astype(jnp.bfloat16)

    def _dotb(v):
        return jax.lax.dot_general(
            v.astype(jnp.bfloat16), expTb, (((1,), (0,)), ((), ())),
            preferred_element_type=jnp.float32)

    rolls = [pltpu.roll(emitb_ref[s, :, :], s, axis=0) for s in range(W)]
    md = rolls[0]
    for s in range(1, W):
        md = jnp.maximum(md, rolls[s])
    md = md + negmask
    for sh in (64, 32, 16, 8, 4, 2, 1):
        md = jnp.maximum(md, pltpu.roll(md, sh, axis=1))
    mb2 = md + pltpu.roll(md, 1, axis=0) + 4.0
    mb2_ref[...] = mb2
    for s in range(W):
        bnd_ref[:, W - 1 - s:W - s, :] = (
            rolls[s] - mb2 + negmask).reshape(L, 1, TP)

    alpha0 = jnp.where(lane == START, 0.0, -1000.0)
    row8 = jax.lax.broadcasted_iota(jnp.int32, (W, TP), 0)
    buf0 = jnp.where(row8 == W - 1,
                     jnp.broadcast_to(alpha0, (W, TP)),
                     NEG)
    r0 = jnp.zeros((1, 1), jnp.float32)

    def step(j, carry):
        buf, na1, r2, mxhi, mxlo = carry
        r1 = jnp.maximum(jnp.max(na1, keepdims=True), NEG)
        ebm = bnd_ref[pl.ds(j, 1), :, :][0]
        mb2row = mb2_ref[pl.ds(j, 1), :]
        ex = (buf - r2) + ebm
        e = jnp.exp(ex)
        ev = jnp.sum(e, axis=0, keepdims=True)
        na = (r2 + jnp.log(_dotb(ev))) + mb2row
        mxv = jnp.max(ex, keepdims=True)
        return (jnp.concatenate([buf[1:], na], axis=0), na, r1,
                jnp.maximum(mxhi, mxv), jnp.minimum(mxlo, mxv))

    big = jnp.full((1, 1), 1e30, jnp.float32)
    alpha0m = jnp.where(lane < NT, alpha0, NEG)
    buf_spec, _, _, mxhi, mxlo = jax.lax.fori_loop(
        0, L, step, (buf0, alpha0m, r0, -big, big))
    bufres_ref[...] = buf_spec

    bad = jnp.logical_not(
        jnp.logical_and(mxhi[0, 0] <= 30.0, mxlo[0, 0] >= -50.0))

    @pl.when(bad)
    def _():
        def stepx(j, bufx):
            ebm = bnd_ref[pl.ds(j, 1), :, :][0]
            mb2row = mb2_ref[pl.ds(j, 1), :]
            tmp = bufx + ebm
            m2 = jnp.maximum(jnp.max(tmp, keepdims=True), NEG)
            e2 = jnp.exp(tmp - m2)
            ev2 = jnp.sum(e2, axis=0, keepdims=True)
            na = (m2 + jnp.log(_dotb(ev2))) + mb2row
            return jnp.concatenate([bufx[1:], na], axis=0)

        bufres_ref[...] = jax.lax.fori_loop(0, L, stepx, buf0)

    buf = bufres_ref[...]

    final = buf[W - 1:W, :] + negmask
    mf = jnp.max(final, keepdims=True)
    logz = mf + jnp.log(jnp.sum(jnp.exp(final - mf), keepdims=True))

    def gold_step(k, acc):
        i = tags_ref[0, k]
        j = tags_ref[1, k]
        p = tags_ref[2, k]
        c = tags_ref[3, k]
        s = j - i
        onb = jnp.logical_and(s >= 0, s < W)
        ss = jnp.where(onb, s, 0)
        row = emitb_ref[pl.ds(ss, 1), pl.ds(i, 1), :][0]
        trow = trans_ref[pl.ds(p, 1), :]
        pm = lane == p
        emit_term = jnp.where(jnp.logical_and(pm, onb), row, 0.0)
        tb_term = jnp.where(jnp.logical_and(pm, jnp.logical_not(onb)),
                            tagb_ref[...], 0.0)
        tr_term = jnp.where(lane == c, trow, 0.0)
        return acc + emit_term + tb_term + tr_term

    acc = jax.lax.fori_loop(0, NSP, gold_step, jnp.zeros((1, TP), jnp.float32))
    gold = jnp.sum(acc, keepdims=True)
    out_ref[...] = jnp.broadcast_to(logz - gold, (8, TP))


def kernel(feats, tags, dense_w, dense_b, wl_w, wl_b, wr_w, wr_b,
           gl_w, gl_b, gr_w, gr_b, tag_w, tag_b, transitions):
    f = jnp.pad(feats[0], ((0, 2 * BL - L), (0, 0)))
    tag_w_p = jnp.zeros((TP, D), jnp.float32).at[:NT].set(tag_w)
    tag_b_p = jnp.zeros((1, TP), jnp.float32).at[0, :NT].set(tag_b)
    trans_p = jnp.zeros((TP, TP), jnp.float32).at[:NT, :NT].set(transitions)
    tags2 = tags[0].astype(jnp.int32).T

    def full(a):
        return pl.BlockSpec(a.shape, lambda i: (0,) * a.ndim)

    b2 = lambda b: b[None, :]
    args1 = (f, dense_w.T, b2(dense_b), wl_w.T, b2(wl_b), wr_w.T, b2(wr_b),
             gl_w.T, b2(gl_b), gr_w.T, b2(gr_b), tag_w_p.T, tag_b_p)
    emitb = pl.pallas_call(
        _band_kernel,
        grid=(2,),
        in_specs=[full(a) for a in args1],
        out_specs=pl.BlockSpec((W, 256, TP), lambda i: (0, i, 0)),
        out_shape=jax.ShapeDtypeStruct((W, L, TP), jnp.float32),
        compiler_params=pltpu.CompilerParams(
            dimension_semantics=("parallel",)),
    )(*args1)

    out = pl.pallas_call(
        _scan_kernel,
        in_specs=[
            pl.BlockSpec(memory_space=pltpu.VMEM),
            pl.BlockSpec(memory_space=pltpu.VMEM),
            pl.BlockSpec(memory_space=pltpu.VMEM),
            pl.BlockSpec(memory_space=pltpu.SMEM),
        ],
        out_specs=pl.BlockSpec(memory_space=pltpu.VMEM),
        out_shape=jax.ShapeDtypeStruct((8, TP), jnp.float32),
        scratch_shapes=[pltpu.VMEM((L, W, TP), jnp.float32),
                        pltpu.VMEM((L, TP), jnp.float32),
                        pltpu.VMEM((W, TP), jnp.float32)],
    )(emitb, trans_p, tag_b_p, tags2)
    return out[0, 0]
